# Initial kernel scaffold; baseline (speedup 1.0000x reference)
import jax
import jax.numpy as jnp
from jax.experimental import pallas as pl


def kernel(inputs, table, pos_encoding):
    raise NotImplementedError("write your pallas kernel here")


# speedup vs baseline: 1.7350x; 1.7350x over previous
import functools

import jax
import jax.numpy as jnp
from jax import lax
from jax.experimental import pallas as pl
from jax.experimental.pallas import tpu as pltpu
from jax.experimental.pallas import tpu_sc as plsc

VOCAB = 100000
D = 64
DP = 128
BATCH = 4096
SEQ = 200
NC, NS = 2, 16
NW = NC * NS
BW = BATCH // NW
SCALE = 8.0


def _sc_body(idxT_hbm, pos_hbm, table_hbm, out_hbm, idx_v, buf_g, buf_o, pos_v, sem):
    wid = lax.axis_index("s") * NC + lax.axis_index("c")
    b0 = wid * BW

    pltpu.sync_copy(pos_hbm, pos_v)

    lanes = lax.iota(jnp.int32, 16)
    ci = [lanes + 16 * c for c in range(4)]

    @pl.loop(0, SEQ)
    def _s(s):
        pltpu.sync_copy(idxT_hbm.at[s, pl.ds(b0, BW)], idx_v)
        pltpu.make_async_copy(table_hbm.at[idx_v], buf_g, sem).start()
        pltpu.make_async_copy(table_hbm.at[idx_v], buf_g, sem).wait()

        pv = [pos_v[s, pl.ds(16 * c, 16)] for c in range(4)]

        @pl.loop(0, BW // 16)
        def _b(bi):
            for u in range(16):
                b = bi * 16 + u
                bvec = jnp.full((16,), 0, jnp.int32) + b
                for c in range(4):
                    x = buf_g[b, pl.ds(16 * c, 16)]
                    plsc.store_scatter(buf_o, [ci[c], bvec], x * SCALE + pv[c])

        pltpu.sync_copy(buf_o, out_hbm.at[s, pl.ds(0, D), pl.ds(b0, BW)])


@jax.jit
def _embed(idxT, posP, tableP):
    mesh = plsc.VectorSubcoreMesh(core_axis_name="c", subcore_axis_name="s")
    kfn = pl.kernel(
        _sc_body,
        out_type=jax.ShapeDtypeStruct((SEQ, D, BATCH), jnp.float32),
        mesh=mesh,
        scratch_types=[
            pltpu.VMEM((BW,), jnp.int32),
            pltpu.VMEM((BW, DP), jnp.float32),
            pltpu.VMEM((D, BW), jnp.float32),
            pltpu.VMEM((SEQ, DP), jnp.float32),
            pltpu.SemaphoreType.DMA,
        ],
        compiler_params=pltpu.CompilerParams(needs_layout_passes=False),
    )
    return kfn(idxT, posP, tableP)


def kernel(inputs, table, pos_encoding):
    idxT = inputs.T
    posP = jnp.pad(pos_encoding[:SEQ], ((0, 0), (0, DP - D)))
    tableP = jnp.pad(table, ((0, 0), (0, DP - D)))
    out = _embed(idxT, posP, tableP)
    return out.transpose(2, 0, 1)


# speedup vs baseline: 2.3205x; 1.3375x over previous
import jax
import jax.numpy as jnp
from jax import lax
from jax.experimental import pallas as pl
from jax.experimental.pallas import tpu as pltpu
from jax.experimental.pallas import tpu_sc as plsc

VOCAB = 100000
D = 64
DP = 128
BATCH = 4096
SEQ = 200
NC, NS = 2, 16
NW = NC * NS
BW = BATCH // NW
SCALE = 8.0


def _sc_body(idxT_hbm, pos_hbm, table_hbm, out_hbm,
             idx_v, buf_g, buf_o, pos_v,
             sem_i0, sem_i1, sem_g0, sem_g1, sem_o0, sem_o1):
    wid = lax.axis_index("s") * NC + lax.axis_index("c")
    b0 = wid * BW
    sem_i = (sem_i0, sem_i1)
    sem_g = (sem_g0, sem_g1)
    sem_o = (sem_o0, sem_o1)

    pltpu.sync_copy(pos_hbm, pos_v)

    lanes = lax.iota(jnp.int32, 16)
    ci = [lanes + 16 * c for c in range(4)]

    def idx_fetch(s, u):
        return pltpu.make_async_copy(
            idxT_hbm.at[s, pl.ds(b0, BW)], idx_v.at[u], sem_i[u])

    def gather(u):
        return pltpu.make_async_copy(
            table_hbm.at[idx_v.at[u]], buf_g.at[u], sem_g[u])

    def writeback(s, u):
        return pltpu.make_async_copy(
            buf_o.at[u], out_hbm.at[s, pl.ds(0, D), pl.ds(b0, BW)], sem_o[u])

    def compute(s, u):
        pv = [pos_v[s, pl.ds(16 * c, 16)] for c in range(4)]

        @pl.loop(0, BW // 16)
        def _b(bi):
            for uu in range(16):
                b = bi * 16 + uu
                bvec = jnp.zeros((16,), jnp.int32) + b
                for c in range(4):
                    x = buf_g[u, b, pl.ds(16 * c, 16)]
                    plsc.store_scatter(
                        buf_o.at[u], [ci[c], bvec], x * SCALE + pv[c])

    for u in range(2):
        idx_fetch(u, u).start()
    for u in range(2):
        idx_fetch(u, u).wait()
        gather(u).start()

    for u in range(2):
        s = u
        gather(u).wait()
        idx_fetch(s + 2, u).start()
        compute(s, u)
        writeback(s, u).start()
        idx_fetch(s + 2, u).wait()
        gather(u).start()

    @pl.loop(2, SEQ, step=2)
    def _s(g):
        for u in range(2):
            s = g + u
            gather(u).wait()

            @pl.when(s + 2 < SEQ)
            def _prefetch_idx():
                idx_fetch(s + 2, u).start()

            writeback(s, u).wait()
            compute(s, u)
            writeback(s, u).start()

            @pl.when(s + 2 < SEQ)
            def _next_gather():
                idx_fetch(s + 2, u).wait()
                gather(u).start()

    for u in range(2):
        writeback(SEQ - 2 + u, u).wait()


@jax.jit
def _embed(idxT, posP, tableP):
    mesh = plsc.VectorSubcoreMesh(core_axis_name="c", subcore_axis_name="s")
    kfn = pl.kernel(
        _sc_body,
        out_type=jax.ShapeDtypeStruct((SEQ, D, BATCH), jnp.float32),
        mesh=mesh,
        scratch_types=[
            pltpu.VMEM((2, BW), jnp.int32),
            pltpu.VMEM((2, BW, DP), jnp.float32),
            pltpu.VMEM((2, D, BW), jnp.float32),
            pltpu.VMEM((SEQ, DP), jnp.float32),
            pltpu.SemaphoreType.DMA,
            pltpu.SemaphoreType.DMA,
            pltpu.SemaphoreType.DMA,
            pltpu.SemaphoreType.DMA,
            pltpu.SemaphoreType.DMA,
            pltpu.SemaphoreType.DMA,
        ],
        compiler_params=pltpu.CompilerParams(needs_layout_passes=False),
    )
    return kfn(idxT, posP, tableP)


def kernel(inputs, table, pos_encoding):
    idxT = inputs.T
    posP = jnp.pad(pos_encoding[:SEQ], ((0, 0), (0, DP - D)))
    tableP = jnp.pad(table, ((0, 0), (0, DP - D)))
    out = _embed(idxT, posP, tableP)
    return out.transpose(2, 0, 1)


# speedup vs baseline: 3.2274x; 1.3908x over previous
import jax
import jax.numpy as jnp
from jax import lax
from jax.experimental import pallas as pl
from jax.experimental.pallas import tpu as pltpu
from jax.experimental.pallas import tpu_sc as plsc

VOCAB = 100000
D = 64
DP = 128
BATCH = 4096
SEQ = 200
NC, NS = 2, 16
NW = NC * NS
BW = BATCH // NW
SCALE = 8.0


def _sc_body(idxT_hbm, pos_hbm, table_hbm, out_hbm,
             idx_v, buf_g, buf_o, pos_v,
             sem_i0, sem_i1, sem_g0, sem_g1, sem_o0, sem_o1):
    wid = lax.axis_index("s") * NC + lax.axis_index("c")
    b0 = wid * BW
    sem_i = (sem_i0, sem_i1)
    sem_g = (sem_g0, sem_g1)
    sem_o = (sem_o0, sem_o1)

    pltpu.sync_copy(pos_hbm, pos_v)

    lanes = lax.iota(jnp.int32, 16)
    ci = [lanes + 16 * c for c in range(4)]

    def idx_fetch(s, u):
        return pltpu.make_async_copy(
            idxT_hbm.at[s, pl.ds(b0, BW)], idx_v.at[u], sem_i[u])

    def gather(u):
        return pltpu.make_async_copy(
            table_hbm.at[idx_v.at[u]], buf_g.at[u], sem_g[u])

    def writeback(s, u):
        return pltpu.make_async_copy(
            buf_o.at[u], out_hbm.at[s, pl.ds(0, D), pl.ds(b0, BW)], sem_o[u])

    rot = [lax.rem(lanes + t, jnp.int32(16)) for t in range(16)]

    def compute(s, u):
        s_vec = jnp.zeros((16,), jnp.int32) + s

        @pl.loop(0, BW // 16)
        def _b(bq):
            bvec = lanes + bq * 16
            for c0 in range(4):
                for t in range(16):
                    cvec = rot[t] + (16 * c0)
                    pvr = plsc.load_gather(pos_v, [s_vec, cvec])
                    x = plsc.load_gather(buf_g.at[u], [bvec, cvec])
                    plsc.store_scatter(
                        buf_o.at[u], [cvec, bvec], x * SCALE + pvr)

    for u in range(2):
        idx_fetch(u, u).start()
    for u in range(2):
        idx_fetch(u, u).wait()
        gather(u).start()

    @pl.loop(0, SEQ, step=2)
    def _s(g):
        for u in range(2):
            s = g + u
            gather(u).wait()

            @pl.when(s + 2 < SEQ)
            def _prefetch_idx():
                idx_fetch(s + 2, u).start()

            @pl.when(s >= 2)
            def _wait_prev_writeback():
                writeback(s, u).wait()

            compute(s, u)
            writeback(s, u).start()

            @pl.when(s + 2 < SEQ)
            def _next_gather():
                idx_fetch(s + 2, u).wait()
                gather(u).start()

    for u in range(2):
        writeback(SEQ - 2 + u, u).wait()


@jax.jit
def _embed(idxT, posP, tableP):
    mesh = plsc.VectorSubcoreMesh(core_axis_name="c", subcore_axis_name="s")
    kfn = pl.kernel(
        _sc_body,
        out_type=jax.ShapeDtypeStruct((SEQ, D, BATCH), jnp.float32),
        mesh=mesh,
        scratch_types=[
            pltpu.VMEM((2, BW), jnp.int32),
            pltpu.VMEM((2, BW, DP), jnp.float32),
            pltpu.VMEM((2, D, BW), jnp.float32),
            pltpu.VMEM((SEQ, DP), jnp.float32),
            pltpu.SemaphoreType.DMA,
            pltpu.SemaphoreType.DMA,
            pltpu.SemaphoreType.DMA,
            pltpu.SemaphoreType.DMA,
            pltpu.SemaphoreType.DMA,
            pltpu.SemaphoreType.DMA,
        ],
        compiler_params=pltpu.CompilerParams(needs_layout_passes=False),
    )
    return kfn(idxT, posP, tableP)


def kernel(inputs, table, pos_encoding):
    idxT = inputs.T
    posP = jnp.pad(pos_encoding[:SEQ], ((0, 0), (0, DP - D)))
    tableP = jnp.pad(table, ((0, 0), (0, DP - D)))
    out = _embed(idxT, posP, tableP)
    return out.transpose(2, 0, 1)
